# Initial kernel scaffold; baseline (speedup 1.0000x reference)
import jax
import jax.numpy as jnp
from jax.experimental import pallas as pl


def kernel(x, edge_index, batch_size, Wl0, bl0, Wr0, Wl1, bl1, Wr1, Wl2, bl2, Wr2, g0, be0, g1, be1, g2, be2, cW1, cb1, cg1, cbe1, cW2, cb2, cg2, cbe2, cW3, cb3):
    raise NotImplementedError("write your pallas kernel here")


# speedup vs baseline: 1.9563x; 1.9563x over previous
import functools
import math

import jax
import jax.numpy as jnp
from jax import lax
from jax.experimental import pallas as pl
from jax.experimental.pallas import tpu as pltpu
from jax.experimental.pallas import tpu_sc as plsc

N, E, D, H = 10000, 320000, 128, 128
B = 1024
BN_EPS = 1e-5
BN_S = 1.0 / math.sqrt(1.0 + BN_EPS)

NC = 2
NS = 16
NW = NC * NS
EP = E // NW
CH = 40
EPP = 10240
NCH = EPP // CH
NPAD = 10240
RPT = NPAD // NS
DUMMY = NPAD - 8

_FP = jnp.float32

_MESH = plsc.VectorSubcoreMesh(core_axis_name="c", subcore_axis_name="s",
                               num_cores=NC, num_subcores=NS)


def _agg_body(count_mode, *args):
    if count_mode:
        dstg, out_hbm, src_v, dst_v, rows, acc, sem = args
    else:
        h_hbm, srcg, dstg, out_hbm, src_v, dst_v, rows, acc, sem = args

    cid = lax.axis_index("c")
    sid = lax.axis_index("s")
    wid = sid * NC + cid
    z16 = jnp.zeros((16,), _FP)

    def zb(i, carry):
        for j in range(H // 16):
            rows[i, pl.ds(j * 16, 16)] = z16
        return carry

    lax.fori_loop(0, CH, zb, 0)
    base = sid * RPT
    for j in range(RPT // CH):
        pltpu.sync_copy(rows.at[pl.ds(0, CH)],
                        acc.at[pl.ds(base + j * CH, CH)])

    if count_mode:
        o16 = jnp.ones((16,), _FP)

        def ob(i, carry):
            for j in range(H // 16):
                rows[i, pl.ds(j * 16, 16)] = o16
            return carry

        lax.fori_loop(0, CH, ob, 0)

    plsc.subcore_barrier()
    ebase = wid * EPP

    def eb(k, carry):
        pltpu.sync_copy(dstg.at[pl.ds(ebase + k * CH, CH)], dst_v)
        if not count_mode:
            pltpu.sync_copy(srcg.at[pl.ds(ebase + k * CH, CH)], src_v)
            pltpu.async_copy(h_hbm.at[src_v], rows, sem).wait()
        pltpu.sync_copy(rows, acc.at[dst_v], add=True)
        return carry

    lax.fori_loop(0, NCH, eb, 0)

    plsc.subcore_barrier()
    pltpu.sync_copy(acc.at[pl.ds(base, RPT)],
                    out_hbm.at[cid, pl.ds(base, RPT)])


def _make_agg(count_mode):
    return pl.kernel(
        functools.partial(_agg_body, count_mode),
        out_type=[jax.ShapeDtypeStruct((NC, NPAD, H), _FP)],
        mesh=_MESH,
        scratch_types=[
            pltpu.VMEM((CH,), jnp.int32),
            pltpu.VMEM((CH,), jnp.int32),
            pltpu.VMEM((CH, H), _FP),
            pltpu.VMEM_SHARED((NPAD, H), _FP),
            pltpu.SemaphoreType.DMA,
        ],
    )


_agg = _make_agg(False)
_count = _make_agg(True)


def _dot(a, b):
    return jnp.dot(a, b, preferred_element_type=_FP)


def _dense_body(parts, h, cntp, Wl, bl, Wr, g, be, out):
    cnt = cntp[0, :, 0:1] + cntp[1, :, 0:1]
    inv = 1.0 / jnp.maximum(cnt, 1.0)
    mean = (parts[0] + parts[1]) * inv
    z = _dot(mean, Wl[...]) + bl[...] + _dot(h[...], Wr[...])
    out[...] = jnp.maximum(z, 0.0) * (g[...] * BN_S) + be[...]


_DR = 2000


def _dense(parts, h, cntp, Wl, bl, Wr, g, be):
    return pl.pallas_call(
        _dense_body,
        grid=(N // _DR,),
        in_specs=[
            pl.BlockSpec((NC, _DR, H), lambda i: (0, i, 0)),
            pl.BlockSpec((_DR, H), lambda i: (i, 0)),
            pl.BlockSpec((NC, _DR, H), lambda i: (0, i, 0)),
            pl.BlockSpec((H, H), lambda i: (0, 0)),
            pl.BlockSpec((1, H), lambda i: (0, 0)),
            pl.BlockSpec((H, H), lambda i: (0, 0)),
            pl.BlockSpec((1, H), lambda i: (0, 0)),
            pl.BlockSpec((1, H), lambda i: (0, 0)),
        ],
        out_specs=pl.BlockSpec((_DR, H), lambda i: (i, 0)),
        out_shape=jax.ShapeDtypeStruct((N, H), _FP),
    )(parts, h, cntp, Wl, bl, Wr, g, be)


def _head_body(parts, h, cntp, Wl, bl, Wr, g, be,
               cW1, cb1, cg1, cbe1, cW2, cb2, cg2, cbe2, cW3, cb3, out):
    cnt = cntp[0, :, 0:1] + cntp[1, :, 0:1]
    inv = 1.0 / jnp.maximum(cnt, 1.0)
    mean = (parts[0] + parts[1]) * inv
    z = _dot(mean, Wl[...]) + bl[...] + _dot(h[...], Wr[...])
    t = jnp.maximum(z, 0.0) * (g[...] * BN_S) + be[...]
    c1 = (_dot(t, cW1[...]) + cb1[...]) * BN_S * cg1[...] + cbe1[...]
    c1 = jnp.maximum(c1, 0.0)
    c2 = (_dot(c1, cW2[...]) + cb2[...]) * BN_S * cg2[...] + cbe2[...]
    c2 = jnp.maximum(c2, 0.0)
    out[...] = _dot(c2, cW3[...]) + cb3[...]


def _head(parts, h, cntp, Wl, bl, Wr, g, be,
          cW1, cb1, cg1, cbe1, cW2, cb2, cg2, cbe2, cW3, cb3):
    H2 = H // 2
    full = lambda *s: pl.BlockSpec(s, lambda i: tuple(0 for _ in s))
    return pl.pallas_call(
        _head_body,
        grid=(1,),
        in_specs=[
            full(NC, B, H), full(B, H), full(NC, B, H),
            full(H, H), full(1, H), full(H, H), full(1, H), full(1, H),
            full(H, H), full(1, H), full(1, H), full(1, H),
            full(H, H2), full(1, H2), full(1, H2), full(1, H2),
            full(H2, 1), full(1, 1),
        ],
        out_specs=full(B, 1),
        out_shape=jax.ShapeDtypeStruct((B, 1), _FP),
    )(parts, h, cntp, Wl, bl, Wr, g, be,
      cW1, cb1, cg1, cbe1, cW2, cb2, cg2, cbe2, cW3, cb3)


def kernel(x, edge_index, batch_size,
           Wl0, bl0, Wr0, Wl1, bl1, Wr1, Wl2, bl2, Wr2,
           g0, be0, g1, be1, g2, be2,
           cW1, cb1, cg1, cbe1, cW2, cb2, cg2, cbe2, cW3, cb3):
    ei = edge_index.astype(jnp.int32)
    pad = ((0, 0), (0, EPP - EP))
    srcg = jnp.pad(ei[0].reshape(NW, EP), pad).reshape(NW * EPP)
    dstg = jnp.pad(ei[1].reshape(NW, EP), pad,
                   constant_values=DUMMY).reshape(NW * EPP)

    r2 = lambda v: v.reshape(1, -1)

    cntp = _count(dstg)[0]
    parts = _agg(x, srcg, dstg)[0]
    h1 = _dense(parts, x, cntp, Wl0, r2(bl0), Wr0, r2(g0), r2(be0))
    parts = _agg(h1, srcg, dstg)[0]
    h2 = _dense(parts, h1, cntp, Wl1, r2(bl1), Wr1, r2(g1), r2(be1))
    parts = _agg(h2, srcg, dstg)[0]
    out = _head(parts[:, :B], h2[:B], cntp[:, :B],
                Wl2, r2(bl2), Wr2, r2(g2), r2(be2),
                cW1, r2(cb1), cg1.reshape(1, H), r2(cbe1),
                cW2, r2(cb2), r2(cg2), r2(cbe2), cW3, cb3.reshape(1, 1))
    return out.reshape(B)


# speedup vs baseline: 3.1286x; 1.5993x over previous
import functools
import math

import jax
import jax.numpy as jnp
from jax import lax
from jax.experimental import pallas as pl
from jax.experimental.pallas import tpu as pltpu
from jax.experimental.pallas import tpu_sc as plsc

N, E, D, H = 10000, 320000, 128, 128
B = 1024
BN_EPS = 1e-5
BN_S = 1.0 / math.sqrt(1.0 + BN_EPS)

NC = 2
NS = 16
NW = NC * NS
EP = E // NW
CH = 40
EPP = 10240
NCH = EPP // CH
NPAD = 10240
RPT = NPAD // NS
DUMMY = NPAD - 8
U = 4
GG = 8
NG = NCH // GG

_FP = jnp.float32

_MESH = plsc.VectorSubcoreMesh(core_axis_name="c", subcore_axis_name="s",
                               num_cores=NC, num_subcores=NS)


def _agg_body(count_mode, *args):
    if count_mode:
        dstg, out_hbm, dst_g, ones_r, acc, *sems = args
    else:
        h_hbm, srcg, dstg, out_hbm, src_g, dst_g, rows, acc, *sems = args
        ones_r = rows.at[0]

    cid = lax.axis_index("c")
    sid = lax.axis_index("s")
    wid = sid * NC + cid
    z16 = jnp.zeros((16,), _FP)

    def zb(i, carry):
        for j in range(H // 16):
            ones_r[i, pl.ds(j * 16, 16)] = z16
        return carry

    lax.fori_loop(0, CH, zb, 0)
    base = sid * RPT
    for j in range(RPT // CH):
        pltpu.sync_copy(ones_r, acc.at[pl.ds(base + j * CH, CH)])

    if count_mode:
        o16 = jnp.ones((16,), _FP)

        def ob(i, carry):
            for j in range(H // 16):
                ones_r[i, pl.ds(j * 16, 16)] = o16
            return carry

        lax.fori_loop(0, CH, ob, 0)

    plsc.subcore_barrier()

    def gb(g, carry):
        pltpu.sync_copy(dstg.at[wid, pl.ds(g * GG, GG)], dst_g)
        if not count_mode:
            pltpu.sync_copy(
                srcg.at[pl.ds(wid * EPP + g * (GG * CH), GG * CH)], src_g)
        for u in range(GG // U):
            if count_mode:
                ds = [pltpu.async_copy(ones_r, acc.at[dst_g.at[u * U + b]],
                                       sems[b], add=True)
                      for b in range(U)]
                for d in ds:
                    d.wait()
            else:
                gs = [pltpu.async_copy(
                    h_hbm.at[src_g.at[pl.ds((u * U + b) * CH, CH)]],
                    rows.at[b], sems[b]) for b in range(U)]
                ss = []
                for b in range(U):
                    gs[b].wait()
                    ss.append(pltpu.async_copy(
                        rows.at[b], acc.at[dst_g.at[u * U + b]],
                        sems[U + b], add=True))
                for d in ss:
                    d.wait()
        return carry

    lax.fori_loop(0, NG, gb, 0)

    plsc.subcore_barrier()
    pltpu.sync_copy(acc.at[pl.ds(base, RPT)],
                    out_hbm.at[cid, pl.ds(base, RPT)])


def _make_agg(count_mode):
    if count_mode:
        scratch = [
            pltpu.VMEM((GG, CH), jnp.int32),
            pltpu.VMEM((CH, H), _FP),
            pltpu.VMEM_SHARED((NPAD, H), _FP),
        ] + [pltpu.SemaphoreType.DMA] * U
    else:
        scratch = [
            pltpu.VMEM((GG * CH,), jnp.int32),
            pltpu.VMEM((GG, CH), jnp.int32),
            pltpu.VMEM((U, CH, H), _FP),
            pltpu.VMEM_SHARED((NPAD, H), _FP),
        ] + [pltpu.SemaphoreType.DMA] * (2 * U)
    return pl.kernel(
        functools.partial(_agg_body, count_mode),
        out_type=[jax.ShapeDtypeStruct((NC, NPAD, H), _FP)],
        mesh=_MESH,
        scratch_types=scratch,
    )


_agg = _make_agg(False)
_count = _make_agg(True)


def _dot(a, b):
    return jnp.dot(a, b, preferred_element_type=_FP)


def _dense_body(parts, h, cntp, Wl, bl, Wr, g, be, out):
    cnt = cntp[0, :, 0:1] + cntp[1, :, 0:1]
    inv = 1.0 / jnp.maximum(cnt, 1.0)
    mean = (parts[0] + parts[1]) * inv
    z = _dot(mean, Wl[...]) + bl[...] + _dot(h[...], Wr[...])
    out[...] = jnp.maximum(z, 0.0) * (g[...] * BN_S) + be[...]


_DR = 2000


def _dense(parts, h, cntp, Wl, bl, Wr, g, be):
    return pl.pallas_call(
        _dense_body,
        grid=(N // _DR,),
        in_specs=[
            pl.BlockSpec((NC, _DR, H), lambda i: (0, i, 0)),
            pl.BlockSpec((_DR, H), lambda i: (i, 0)),
            pl.BlockSpec((NC, _DR, H), lambda i: (0, i, 0)),
            pl.BlockSpec((H, H), lambda i: (0, 0)),
            pl.BlockSpec((1, H), lambda i: (0, 0)),
            pl.BlockSpec((H, H), lambda i: (0, 0)),
            pl.BlockSpec((1, H), lambda i: (0, 0)),
            pl.BlockSpec((1, H), lambda i: (0, 0)),
        ],
        out_specs=pl.BlockSpec((_DR, H), lambda i: (i, 0)),
        out_shape=jax.ShapeDtypeStruct((N, H), _FP),
    )(parts, h, cntp, Wl, bl, Wr, g, be)


def _head_body(parts, h, cntp, Wl, bl, Wr, g, be,
               cW1, cb1, cg1, cbe1, cW2, cb2, cg2, cbe2, cW3, cb3, out):
    cnt = cntp[0, :, 0:1] + cntp[1, :, 0:1]
    inv = 1.0 / jnp.maximum(cnt, 1.0)
    mean = (parts[0] + parts[1]) * inv
    z = _dot(mean, Wl[...]) + bl[...] + _dot(h[...], Wr[...])
    t = jnp.maximum(z, 0.0) * (g[...] * BN_S) + be[...]
    c1 = (_dot(t, cW1[...]) + cb1[...]) * BN_S * cg1[...] + cbe1[...]
    c1 = jnp.maximum(c1, 0.0)
    c2 = (_dot(c1, cW2[...]) + cb2[...]) * BN_S * cg2[...] + cbe2[...]
    c2 = jnp.maximum(c2, 0.0)
    out[...] = _dot(c2, cW3[...]) + cb3[...]


def _head(parts, h, cntp, Wl, bl, Wr, g, be,
          cW1, cb1, cg1, cbe1, cW2, cb2, cg2, cbe2, cW3, cb3):
    H2 = H // 2
    full = lambda *s: pl.BlockSpec(s, lambda i: tuple(0 for _ in s))
    return pl.pallas_call(
        _head_body,
        grid=(1,),
        in_specs=[
            full(NC, B, H), full(B, H), full(NC, B, H),
            full(H, H), full(1, H), full(H, H), full(1, H), full(1, H),
            full(H, H), full(1, H), full(1, H), full(1, H),
            full(H, H2), full(1, H2), full(1, H2), full(1, H2),
            full(H2, 1), full(1, 1),
        ],
        out_specs=full(B, 1),
        out_shape=jax.ShapeDtypeStruct((B, 1), _FP),
    )(parts, h, cntp, Wl, bl, Wr, g, be,
      cW1, cb1, cg1, cbe1, cW2, cb2, cg2, cbe2, cW3, cb3)


def kernel(x, edge_index, batch_size,
           Wl0, bl0, Wr0, Wl1, bl1, Wr1, Wl2, bl2, Wr2,
           g0, be0, g1, be1, g2, be2,
           cW1, cb1, cg1, cbe1, cW2, cb2, cg2, cbe2, cW3, cb3):
    ei = edge_index.astype(jnp.int32)
    pad = ((0, 0), (0, EPP - EP))
    srcg = jnp.pad(ei[0].reshape(NW, EP), pad).reshape(NW * EPP)
    dstg = jnp.pad(ei[1].reshape(NW, EP), pad,
                   constant_values=DUMMY).reshape(NW, NCH, CH)

    r2 = lambda v: v.reshape(1, -1)

    cntp = _count(dstg)[0]
    parts = _agg(x, srcg, dstg)[0]
    h1 = _dense(parts, x, cntp, Wl0, r2(bl0), Wr0, r2(g0), r2(be0))
    parts = _agg(h1, srcg, dstg)[0]
    h2 = _dense(parts, h1, cntp, Wl1, r2(bl1), Wr1, r2(g1), r2(be1))
    parts = _agg(h2, srcg, dstg)[0]
    out = _head(parts[:, :B], h2[:B], cntp[:, :B],
                Wl2, r2(bl2), Wr2, r2(g2), r2(be2),
                cW1, r2(cb1), cg1.reshape(1, H), r2(cbe1),
                cW2, r2(cb2), r2(cg2), r2(cbe2), cW3, cb3.reshape(1, 1))
    return out.reshape(B)


# speedup vs baseline: 3.4463x; 1.1015x over previous
import functools
import math

import jax
import jax.numpy as jnp
from jax import lax
from jax.experimental import pallas as pl
from jax.experimental.pallas import tpu as pltpu
from jax.experimental.pallas import tpu_sc as plsc

N, E, D, H = 10000, 320000, 128, 128
B = 1024
BN_EPS = 1e-5
BN_S = 1.0 / math.sqrt(1.0 + BN_EPS)

NC = 2
NS = 16
NW = NC * NS
EP = E // NW
EPP = 10240
CHA = 64
CHC = 40
NPAD = 10240
RPT = NPAD // NS
DUMMY = NPAD - 8
U = 4
GG = 8

_FP = jnp.float32

_MESH = plsc.VectorSubcoreMesh(core_axis_name="c", subcore_axis_name="s",
                               num_cores=NC, num_subcores=NS)


def _agg_body(count_mode, *args):
    ch = CHC if count_mode else CHA
    ng = EPP // ch // GG
    if count_mode:
        dstg, out_hbm, dst_g, ones_r, acc, *sems = args
    else:
        h_hbm, srcg, dstg, out_hbm, src_g, dst_g, rows, acc, *sems = args
        ones_r = rows.at[0]

    cid = lax.axis_index("c")
    sid = lax.axis_index("s")
    wid = sid * NC + cid
    z16 = jnp.zeros((16,), _FP)

    def zb(i, carry):
        for j in range(H // 16):
            ones_r[i, pl.ds(j * 16, 16)] = z16
        return carry

    lax.fori_loop(0, ch, zb, 0)
    base = sid * RPT
    for j in range(RPT // ch):
        pltpu.sync_copy(ones_r, acc.at[pl.ds(base + j * ch, ch)])

    if count_mode:
        o16 = jnp.ones((16,), _FP)

        def ob(i, carry):
            for j in range(H // 16):
                ones_r[i, pl.ds(j * 16, 16)] = o16
            return carry

        lax.fori_loop(0, ch, ob, 0)

    plsc.subcore_barrier()

    def gb(g, carry):
        pltpu.sync_copy(dstg.at[wid, pl.ds(g * GG, GG)], dst_g)
        if count_mode:
            ds = [pltpu.async_copy(ones_r, acc.at[dst_g.at[k]],
                                   sems[k % U], add=True)
                  for k in range(GG)]
            for d in ds:
                d.wait()
            return carry

        pltpu.sync_copy(
            srcg.at[pl.ds(wid * EPP + g * (GG * ch), GG * ch)], src_g)

        def g_start(k):
            return pltpu.async_copy(
                h_hbm.at[src_g.at[pl.ds(k * ch, ch)]],
                rows.at[k % U], sems[k % U])

        gd = [None] * GG
        sd = [None] * GG
        for k in range(U):
            gd[k] = g_start(k)
        for k in range(GG):
            gd[k].wait()
            sd[k] = pltpu.async_copy(rows.at[k % U], acc.at[dst_g.at[k]],
                                     sems[U + k % U], add=True)
            if k + U < GG:
                sd[k].wait()
                gd[k + U] = g_start(k + U)
        for k in range(GG - U, GG):
            sd[k].wait()
        return carry

    lax.fori_loop(0, ng, gb, 0)

    plsc.subcore_barrier()
    pltpu.sync_copy(acc.at[pl.ds(base, RPT)],
                    out_hbm.at[cid, pl.ds(base, RPT)])


def _make_agg(count_mode):
    if count_mode:
        scratch = [
            pltpu.VMEM((GG, CHC), jnp.int32),
            pltpu.VMEM((CHC, H), _FP),
            pltpu.VMEM_SHARED((NPAD, H), _FP),
        ] + [pltpu.SemaphoreType.DMA] * U
    else:
        scratch = [
            pltpu.VMEM((GG * CHA,), jnp.int32),
            pltpu.VMEM((GG, CHA), jnp.int32),
            pltpu.VMEM((U, CHA, H), _FP),
            pltpu.VMEM_SHARED((NPAD, H), _FP),
        ] + [pltpu.SemaphoreType.DMA] * (2 * U)
    return pl.kernel(
        functools.partial(_agg_body, count_mode),
        out_type=[jax.ShapeDtypeStruct((NC, NPAD, H), _FP)],
        mesh=_MESH,
        scratch_types=scratch,
    )


_agg = _make_agg(False)
_count = _make_agg(True)


def _dot(a, b):
    return jnp.dot(a, b, preferred_element_type=_FP)


def _dense_body(parts, h, cntp, Wl, bl, Wr, g, be, out):
    cnt = cntp[0, :, 0:1] + cntp[1, :, 0:1]
    inv = 1.0 / jnp.maximum(cnt, 1.0)
    mean = (parts[0] + parts[1]) * inv
    z = _dot(mean, Wl[...]) + bl[...] + _dot(h[...], Wr[...])
    out[...] = jnp.maximum(z, 0.0) * (g[...] * BN_S) + be[...]


_DR = 2000


def _dense(parts, h, cntp, Wl, bl, Wr, g, be):
    return pl.pallas_call(
        _dense_body,
        grid=(N // _DR,),
        in_specs=[
            pl.BlockSpec((NC, _DR, H), lambda i: (0, i, 0)),
            pl.BlockSpec((_DR, H), lambda i: (i, 0)),
            pl.BlockSpec((NC, _DR, H), lambda i: (0, i, 0)),
            pl.BlockSpec((H, H), lambda i: (0, 0)),
            pl.BlockSpec((1, H), lambda i: (0, 0)),
            pl.BlockSpec((H, H), lambda i: (0, 0)),
            pl.BlockSpec((1, H), lambda i: (0, 0)),
            pl.BlockSpec((1, H), lambda i: (0, 0)),
        ],
        out_specs=pl.BlockSpec((_DR, H), lambda i: (i, 0)),
        out_shape=jax.ShapeDtypeStruct((N, H), _FP),
    )(parts, h, cntp, Wl, bl, Wr, g, be)


def _head_body(parts, h, cntp, Wl, bl, Wr, g, be,
               cW1, cb1, cg1, cbe1, cW2, cb2, cg2, cbe2, cW3, cb3, out):
    cnt = cntp[0, :, 0:1] + cntp[1, :, 0:1]
    inv = 1.0 / jnp.maximum(cnt, 1.0)
    mean = (parts[0] + parts[1]) * inv
    z = _dot(mean, Wl[...]) + bl[...] + _dot(h[...], Wr[...])
    t = jnp.maximum(z, 0.0) * (g[...] * BN_S) + be[...]
    c1 = (_dot(t, cW1[...]) + cb1[...]) * BN_S * cg1[...] + cbe1[...]
    c1 = jnp.maximum(c1, 0.0)
    c2 = (_dot(c1, cW2[...]) + cb2[...]) * BN_S * cg2[...] + cbe2[...]
    c2 = jnp.maximum(c2, 0.0)
    out[...] = _dot(c2, cW3[...]) + cb3[...]


def _head(parts, h, cntp, Wl, bl, Wr, g, be,
          cW1, cb1, cg1, cbe1, cW2, cb2, cg2, cbe2, cW3, cb3):
    H2 = H // 2
    full = lambda *s: pl.BlockSpec(s, lambda i: tuple(0 for _ in s))
    return pl.pallas_call(
        _head_body,
        grid=(1,),
        in_specs=[
            full(NC, B, H), full(B, H), full(NC, B, H),
            full(H, H), full(1, H), full(H, H), full(1, H), full(1, H),
            full(H, H), full(1, H), full(1, H), full(1, H),
            full(H, H2), full(1, H2), full(1, H2), full(1, H2),
            full(H2, 1), full(1, 1),
        ],
        out_specs=full(B, 1),
        out_shape=jax.ShapeDtypeStruct((B, 1), _FP),
    )(parts, h, cntp, Wl, bl, Wr, g, be,
      cW1, cb1, cg1, cbe1, cW2, cb2, cg2, cbe2, cW3, cb3)


def kernel(x, edge_index, batch_size,
           Wl0, bl0, Wr0, Wl1, bl1, Wr1, Wl2, bl2, Wr2,
           g0, be0, g1, be1, g2, be2,
           cW1, cb1, cg1, cbe1, cW2, cb2, cg2, cbe2, cW3, cb3):
    ei = edge_index.astype(jnp.int32)
    pad = ((0, 0), (0, EPP - EP))
    srcg = jnp.pad(ei[0].reshape(NW, EP), pad).reshape(NW * EPP)
    dstp = jnp.pad(ei[1].reshape(NW, EP), pad, constant_values=DUMMY)
    dstg_a = dstp.reshape(NW, EPP // CHA, CHA)
    dstg_c = dstp.reshape(NW, EPP // CHC, CHC)

    r2 = lambda v: v.reshape(1, -1)

    cntp = _count(dstg_c)[0]
    parts = _agg(x, srcg, dstg_a)[0]
    h1 = _dense(parts, x, cntp, Wl0, r2(bl0), Wr0, r2(g0), r2(be0))
    parts = _agg(h1, srcg, dstg_a)[0]
    h2 = _dense(parts, h1, cntp, Wl1, r2(bl1), Wr1, r2(g1), r2(be1))
    parts = _agg(h2, srcg, dstg_a)[0]
    out = _head(parts[:, :B], h2[:B], cntp[:, :B],
                Wl2, r2(bl2), Wr2, r2(g2), r2(be2),
                cW1, r2(cb1), cg1.reshape(1, H), r2(cbe1),
                cW2, r2(cb2), r2(cg2), r2(cbe2), cW3, cb3.reshape(1, 1))
    return out.reshape(B)


# speedup vs baseline: 3.5462x; 1.0290x over previous
import functools
import math

import jax
import jax.numpy as jnp
from jax import lax
from jax.experimental import pallas as pl
from jax.experimental.pallas import tpu as pltpu
from jax.experimental.pallas import tpu_sc as plsc

N, E, D, H = 10000, 320000, 128, 128
B = 1024
BN_EPS = 1e-5
BN_S = 1.0 / math.sqrt(1.0 + BN_EPS)

NC = 2
NS = 16
NW = NC * NS
EP = E // NW
EPP = 10240
CHA = 128
CHC = 40
NPAD = 10240
RPT = NPAD // NS
DUMMY = NPAD - 8
U = 2
GG = 8

_FP = jnp.float32

_MESH = plsc.VectorSubcoreMesh(core_axis_name="c", subcore_axis_name="s",
                               num_cores=NC, num_subcores=NS)


def _agg_body(count_mode, *args):
    ch = CHC if count_mode else CHA
    ng = EPP // ch // GG
    if count_mode:
        dstg, out_hbm, dst_g, ones_r, acc, *sems = args
    else:
        h_hbm, srcg, dstg, out_hbm, src_g, dst_g, rows, acc, *sems = args
        ones_r = rows.at[0]

    cid = lax.axis_index("c")
    sid = lax.axis_index("s")
    wid = sid * NC + cid
    z16 = jnp.zeros((16,), _FP)

    def zb(i, carry):
        for j in range(H // 16):
            ones_r[i, pl.ds(j * 16, 16)] = z16
        return carry

    lax.fori_loop(0, ch, zb, 0)
    base = sid * RPT
    for j in range(RPT // ch):
        pltpu.sync_copy(ones_r, acc.at[pl.ds(base + j * ch, ch)])

    if count_mode:
        o16 = jnp.ones((16,), _FP)

        def ob(i, carry):
            for j in range(H // 16):
                ones_r[i, pl.ds(j * 16, 16)] = o16
            return carry

        lax.fori_loop(0, ch, ob, 0)

    plsc.subcore_barrier()

    def gb(g, carry):
        pltpu.sync_copy(dstg.at[wid, pl.ds(g * GG, GG)], dst_g)
        if count_mode:
            ds = [pltpu.async_copy(ones_r, acc.at[dst_g.at[k]],
                                   sems[k % U], add=True)
                  for k in range(GG)]
            for d in ds:
                d.wait()
            return carry

        pltpu.sync_copy(
            srcg.at[pl.ds(wid * EPP + g * (GG * ch), GG * ch)], src_g)

        def g_start(k):
            return pltpu.async_copy(
                h_hbm.at[src_g.at[pl.ds(k * ch, ch)]],
                rows.at[k % U], sems[k % U])

        gd = [None] * GG
        sd = [None] * GG
        for k in range(U):
            gd[k] = g_start(k)
        for k in range(GG):
            gd[k].wait()
            sd[k] = pltpu.async_copy(rows.at[k % U], acc.at[dst_g.at[k]],
                                     sems[U + k % U], add=True)
            if k + U < GG:
                sd[k].wait()
                gd[k + U] = g_start(k + U)
        for k in range(GG - U, GG):
            sd[k].wait()
        return carry

    lax.fori_loop(0, ng, gb, 0)

    plsc.subcore_barrier()
    pltpu.sync_copy(acc.at[pl.ds(base, RPT)],
                    out_hbm.at[cid, pl.ds(base, RPT)])


def _make_agg(count_mode):
    if count_mode:
        scratch = [
            pltpu.VMEM((GG, CHC), jnp.int32),
            pltpu.VMEM((CHC, H), _FP),
            pltpu.VMEM_SHARED((NPAD, H), _FP),
        ] + [pltpu.SemaphoreType.DMA] * U
    else:
        scratch = [
            pltpu.VMEM((GG * CHA,), jnp.int32),
            pltpu.VMEM((GG, CHA), jnp.int32),
            pltpu.VMEM((U, CHA, H), _FP),
            pltpu.VMEM_SHARED((NPAD, H), _FP),
        ] + [pltpu.SemaphoreType.DMA] * (2 * U)
    return pl.kernel(
        functools.partial(_agg_body, count_mode),
        out_type=[jax.ShapeDtypeStruct((NC, NPAD, H), _FP)],
        mesh=_MESH,
        scratch_types=scratch,
    )


_agg = _make_agg(False)
_count = _make_agg(True)


def _dot(a, b):
    return jnp.dot(a, b, preferred_element_type=_FP)


def _dense_body(parts, h, cntp, Wl, bl, Wr, g, be, out):
    cnt = cntp[0, :, 0:1] + cntp[1, :, 0:1]
    inv = 1.0 / jnp.maximum(cnt, 1.0)
    mean = (parts[0] + parts[1]) * inv
    z = _dot(mean, Wl[...]) + bl[...] + _dot(h[...], Wr[...])
    out[...] = jnp.maximum(z, 0.0) * (g[...] * BN_S) + be[...]


_DR = 2000


def _dense(parts, h, cntp, Wl, bl, Wr, g, be):
    return pl.pallas_call(
        _dense_body,
        grid=(N // _DR,),
        in_specs=[
            pl.BlockSpec((NC, _DR, H), lambda i: (0, i, 0)),
            pl.BlockSpec((_DR, H), lambda i: (i, 0)),
            pl.BlockSpec((NC, _DR, H), lambda i: (0, i, 0)),
            pl.BlockSpec((H, H), lambda i: (0, 0)),
            pl.BlockSpec((1, H), lambda i: (0, 0)),
            pl.BlockSpec((H, H), lambda i: (0, 0)),
            pl.BlockSpec((1, H), lambda i: (0, 0)),
            pl.BlockSpec((1, H), lambda i: (0, 0)),
        ],
        out_specs=pl.BlockSpec((_DR, H), lambda i: (i, 0)),
        out_shape=jax.ShapeDtypeStruct((N, H), _FP),
    )(parts, h, cntp, Wl, bl, Wr, g, be)


def _head_body(parts, h, cntp, Wl, bl, Wr, g, be,
               cW1, cb1, cg1, cbe1, cW2, cb2, cg2, cbe2, cW3, cb3, out):
    cnt = cntp[0, :, 0:1] + cntp[1, :, 0:1]
    inv = 1.0 / jnp.maximum(cnt, 1.0)
    mean = (parts[0] + parts[1]) * inv
    z = _dot(mean, Wl[...]) + bl[...] + _dot(h[...], Wr[...])
    t = jnp.maximum(z, 0.0) * (g[...] * BN_S) + be[...]
    c1 = (_dot(t, cW1[...]) + cb1[...]) * BN_S * cg1[...] + cbe1[...]
    c1 = jnp.maximum(c1, 0.0)
    c2 = (_dot(c1, cW2[...]) + cb2[...]) * BN_S * cg2[...] + cbe2[...]
    c2 = jnp.maximum(c2, 0.0)
    out[...] = _dot(c2, cW3[...]) + cb3[...]


def _head(parts, h, cntp, Wl, bl, Wr, g, be,
          cW1, cb1, cg1, cbe1, cW2, cb2, cg2, cbe2, cW3, cb3):
    H2 = H // 2
    full = lambda *s: pl.BlockSpec(s, lambda i: tuple(0 for _ in s))
    return pl.pallas_call(
        _head_body,
        grid=(1,),
        in_specs=[
            full(NC, B, H), full(B, H), full(NC, B, H),
            full(H, H), full(1, H), full(H, H), full(1, H), full(1, H),
            full(H, H), full(1, H), full(1, H), full(1, H),
            full(H, H2), full(1, H2), full(1, H2), full(1, H2),
            full(H2, 1), full(1, 1),
        ],
        out_specs=full(B, 1),
        out_shape=jax.ShapeDtypeStruct((B, 1), _FP),
    )(parts, h, cntp, Wl, bl, Wr, g, be,
      cW1, cb1, cg1, cbe1, cW2, cb2, cg2, cbe2, cW3, cb3)


def kernel(x, edge_index, batch_size,
           Wl0, bl0, Wr0, Wl1, bl1, Wr1, Wl2, bl2, Wr2,
           g0, be0, g1, be1, g2, be2,
           cW1, cb1, cg1, cbe1, cW2, cb2, cg2, cbe2, cW3, cb3):
    ei = edge_index.astype(jnp.int32)
    pad = ((0, 0), (0, EPP - EP))
    srcg = jnp.pad(ei[0].reshape(NW, EP), pad).reshape(NW * EPP)
    dstp = jnp.pad(ei[1].reshape(NW, EP), pad, constant_values=DUMMY)
    dstg_a = dstp.reshape(NW, EPP // CHA, CHA)
    dstg_c = dstp.reshape(NW, EPP // CHC, CHC)

    r2 = lambda v: v.reshape(1, -1)

    cntp = _count(dstg_c)[0]
    parts = _agg(x, srcg, dstg_a)[0]
    h1 = _dense(parts, x, cntp, Wl0, r2(bl0), Wr0, r2(g0), r2(be0))
    parts = _agg(h1, srcg, dstg_a)[0]
    h2 = _dense(parts, h1, cntp, Wl1, r2(bl1), Wr1, r2(g1), r2(be1))
    parts = _agg(h2, srcg, dstg_a)[0]
    out = _head(parts[:, :B], h2[:B], cntp[:, :B],
                Wl2, r2(bl2), Wr2, r2(g2), r2(be2),
                cW1, r2(cb1), cg1.reshape(1, H), r2(cbe1),
                cW2, r2(cb2), r2(cg2), r2(cbe2), cW3, cb3.reshape(1, 1))
    return out.reshape(B)
